# baseline (speedup 1.0000x reference)
import jax
import jax.numpy as jnp
from jax.experimental import pallas as pl
from jax.experimental.pallas import tpu as pltpu

B = 128
D = 768
V = 27623
K = 256
VB = 1024
NB = 27
VP = NB * VB
LAST = NB - 1

_DN = (((1,), (1,)), ((), ()))


def _softplus(x):
    return jnp.maximum(x, 0.0) + jnp.log1p(jnp.exp(-jnp.abs(x)))


def _masked_u(j, u):
    lane = jax.lax.broadcasted_iota(jnp.int32, (B, VB), 1) + j * VB
    return jnp.where(lane < V, u, 0.0)


def _bisect_steps(j, u_ref, lo_ref, hi_ref):
    n_it = jnp.where(j < 4, 2, 1)

    def body(_, carry):
        lo, hi = carry
        mid = lo + (hi - lo) // 2
        t = jax.lax.bitcast_convert_type(mid, jnp.float32)
        cnt = jnp.sum((u_ref[...] >= t).astype(jnp.int32), axis=1,
                      keepdims=True)
        ge = cnt >= K
        return jnp.where(ge, mid, lo), jnp.where(ge, hi, mid)

    lo, hi = jax.lax.fori_loop(0, n_it, body, (lo_ref[...], hi_ref[...]))
    lo_ref[...] = lo
    hi_ref[...] = hi


def _fused_kernel(h_ref, wp_ref, bp_ref, wm_ref, bm_ref, sr_ref, wd_ref,
                  z_ref, sq_ref, dsp_ref, dsm_ref,
                  up_s, um_s, lop, hip, lom, him, zacc_ref):
    p = pl.program_id(0)
    j = pl.program_id(1)
    sl = (slice(None), pl.ds(j * VB, VB))

    @pl.when(p == 0)
    def _():
        s = jax.lax.dot_general(h_ref[...], wp_ref[...], _DN,
                                preferred_element_type=jnp.float32) + bp_ref[...]
        u = _softplus(s)
        up_s[sl] = jnp.where(j == LAST, _masked_u(j, u), u)

        @pl.when(j == 0)
        def _():
            lop[...] = jnp.zeros((B, 1), jnp.int32)
            hip[...] = jnp.full((B, 1), 0x7F800000, jnp.int32)
            lom[...] = jnp.zeros((B, 1), jnp.int32)
            him[...] = jnp.full((B, 1), 0x7F800000, jnp.int32)
            zacc_ref[...] = jnp.zeros_like(zacc_ref)

    @pl.when(p == 1)
    def _():
        s = jax.lax.dot_general(h_ref[...], wm_ref[...], _DN,
                                preferred_element_type=jnp.float32) + bm_ref[...]
        u = _softplus(s)
        um_s[sl] = jnp.where(j == LAST, _masked_u(j, u), u)
        _bisect_steps(j, up_s, lop, hip)

    @pl.when(p == 2)
    def _():
        tp = jax.lax.bitcast_convert_type(lop[...], jnp.float32)
        u_p = up_s[sl]
        dsp = jnp.where(u_p >= tp, u_p, 0.0)
        dsp_ref[...] = dsp
        sr = sr_ref[...]
        sr = jnp.where(j == LAST, _masked_u(j, sr), sr)
        up_s[sl] = jnp.maximum(sr + dsp, 0.0)
        _bisect_steps(j, um_s, lom, him)

    @pl.when(p == 3)
    def _():
        tm = jax.lax.bitcast_convert_type(lom[...], jnp.float32)
        u_m = um_s[sl]
        dsm = jnp.where(u_m >= tm, u_m, 0.0)
        dsm_ref[...] = dsm
        sq = up_s[sl] - dsm
        sq_ref[...] = sq
        wd = wd_ref[...]
        wlane = jax.lax.broadcasted_iota(jnp.int32, (D, VB), 1) + j * VB
        wd = jnp.where(jnp.logical_and(j == LAST, wlane >= V), 0.0, wd)
        zacc_ref[...] += jax.lax.dot_general(sq, wd, _DN,
                                             preferred_element_type=jnp.float32)

        @pl.when(j == LAST)
        def _():
            z = zacc_ref[...]
            n = jnp.sqrt(jnp.sum(z * z, axis=1, keepdims=True))
            z_ref[...] = z / (n + 1e-6)


def kernel(h_t, sr_plus, sr_minus, W_plus, b_plus, W_minus, b_minus, W_dec):
    bp = b_plus[None, :]
    bm = b_minus[None, :]

    z_hat, sq, ds_plus, ds_minus = pl.pallas_call(
        _fused_kernel,
        grid=(4, NB),
        in_specs=[
            pl.BlockSpec((B, D), lambda p, j: (0, 0)),
            pl.BlockSpec((VB, D), lambda p, j: (jnp.where(p == 0, j, LAST), 0)),
            pl.BlockSpec((1, VB), lambda p, j: (0, jnp.where(p == 0, j, LAST))),
            pl.BlockSpec((VB, D),
                         lambda p, j: (jnp.where(p < 1, 0,
                                                 jnp.where(p == 1, j, LAST)), 0)),
            pl.BlockSpec((1, VB),
                         lambda p, j: (0, jnp.where(p < 1, 0,
                                                    jnp.where(p == 1, j, LAST)))),
            pl.BlockSpec((B, VB),
                         lambda p, j: (0, jnp.where(p < 2, 0,
                                                    jnp.where(p == 2, j, LAST)))),
            pl.BlockSpec((D, VB),
                         lambda p, j: (0, jnp.where(p < 3, 0, j))),
        ],
        out_specs=[
            pl.BlockSpec((B, D), lambda p, j: (0, 0)),
            pl.BlockSpec((B, VB),
                         lambda p, j: (0, jnp.where(p < 3, 0, j))),
            pl.BlockSpec((B, VB),
                         lambda p, j: (0, jnp.where(p < 2, 0,
                                                    jnp.where(p == 2, j, LAST)))),
            pl.BlockSpec((B, VB),
                         lambda p, j: (0, jnp.where(p < 3, 0, j))),
        ],
        out_shape=[
            jax.ShapeDtypeStruct((B, D), jnp.float32),
            jax.ShapeDtypeStruct((B, V), jnp.float32),
            jax.ShapeDtypeStruct((B, V), jnp.float32),
            jax.ShapeDtypeStruct((B, V), jnp.float32),
        ],
        scratch_shapes=[
            pltpu.VMEM((B, VP), jnp.float32),
            pltpu.VMEM((B, VP), jnp.float32),
            pltpu.VMEM((B, 1), jnp.int32),
            pltpu.VMEM((B, 1), jnp.int32),
            pltpu.VMEM((B, 1), jnp.int32),
            pltpu.VMEM((B, 1), jnp.int32),
            pltpu.VMEM((B, D), jnp.float32),
        ],
    )(h_t, W_plus, bp, W_minus, bm, sr_plus, W_dec)

    return (z_hat, sq, ds_plus, ds_minus)
